# baseline (speedup 1.0000x reference)
import functools

import jax
import jax.numpy as jnp
from jax import lax
from jax.experimental import pallas as pl
from jax.experimental.pallas import tpu as pltpu
from jax.experimental.pallas import tpu_sc as plsc

B, S, NCAT, V, ED, CD, DM = 4096, 50, 26, 100000, 32, 16, 128
T = B * S
NC4 = 28
RC = NC4 // 4
N2 = T * NC4
RPT = 128
NW = 32
BLOCKS = N2 // RPT
BPW = BLOCKS // NW
K = 14
OUTER = BPW // K


def _sc_gather(tables_flat, idx_flat):
    mesh = plsc.VectorSubcoreMesh(core_axis_name="c", subcore_axis_name="s")

    @functools.partial(
        pl.kernel,
        out_type=jax.ShapeDtypeStruct((N2, ED), jnp.float32),
        mesh=mesh,
        scratch_types=[
            pltpu.VMEM((K * RPT,), jnp.int32),
            pltpu.VMEM((K * RPT, ED), jnp.float32),
            pltpu.SemaphoreType.DMA,
        ],
        compiler_params=pltpu.CompilerParams(use_tc_tiling_on_sc=False),
    )
    def gather_kernel(tbl_hbm, idx_hbm, out_hbm, idx_v, rows_v, sem):
        wid = lax.axis_index("s") * 2 + lax.axis_index("c")
        wbase = wid * BPW

        def body(g, carry):
            row0 = (wbase + g * K) * RPT
            pltpu.sync_copy(idx_hbm.at[pl.ds(row0, K * RPT)], idx_v)
            copies = []
            for j in range(K):
                copies.append(
                    pltpu.async_copy(
                        tbl_hbm.at[idx_v.at[pl.ds(j * RPT, RPT)]],
                        rows_v.at[pl.ds(j * RPT, RPT)],
                        sem,
                    )
                )
            for c in copies:
                c.wait()
            pltpu.sync_copy(rows_v, out_hbm.at[pl.ds(row0, K * RPT)])
            return carry

        lax.fori_loop(0, OUTER, body, 0)

    return gather_kernel(tables_flat, idx_flat)


def _proj_body(x_ref, w3_ref, n_ref, m_ref, wn_ref, b_ref, o_ref):
    r = pl.program_id(1)

    @pl.when(r == 0)
    def _init():
        o_ref[...] = (
            jnp.dot(n_ref[...] * m_ref[...], wn_ref[...],
                    preferred_element_type=jnp.float32)
            + b_ref[...]
        )

    o_ref[...] += jnp.dot(x_ref[0], w3_ref[0],
                          preferred_element_type=jnp.float32)


def _tc_project(x3, w3, num_inputs, num_mask, Wn, b2):
    BT = 2048
    return pl.pallas_call(
        _proj_body,
        grid=(T // BT, RC),
        in_specs=[
            pl.BlockSpec((1, BT, 128), lambda i, r: (r, i, 0)),
            pl.BlockSpec((1, 128, DM), lambda i, r: (r, 0, 0)),
            pl.BlockSpec((BT, CD), lambda i, r: (i, 0)),
            pl.BlockSpec((BT, CD), lambda i, r: (i, 0)),
            pl.BlockSpec((CD, DM), lambda i, r: (0, 0)),
            pl.BlockSpec((1, DM), lambda i, r: (0, 0)),
        ],
        out_specs=pl.BlockSpec((BT, DM), lambda i, r: (i, 0)),
        out_shape=jax.ShapeDtypeStruct((T, DM), jnp.float32),
    )(x3, w3, num_inputs, num_mask, Wn, b2)


def kernel(cat_inputs, cat_mask, num_inputs, num_mask, tables, W, b):
    tables_flat = tables.reshape(NCAT * V, ED)
    offs = (jnp.arange(NCAT, dtype=jnp.int32) * V)[None, None, :]
    idx26 = (cat_inputs + offs).reshape(T, NCAT)
    idx28 = jnp.concatenate(
        [idx26, jnp.zeros((T, NC4 - NCAT), jnp.int32)], axis=-1)
    idx_flat = idx28.reshape(T, RC, 4).transpose(1, 0, 2).reshape(N2)

    rows = _sc_gather(tables_flat, idx_flat)
    x3 = rows.reshape(RC, T, 128)

    w3 = jnp.pad(W[: NCAT * ED], ((0, RC * 128 - NCAT * ED), (0, 0)))
    w3 = w3.reshape(RC, 128, DM)

    out = _tc_project(
        x3,
        w3,
        num_inputs.reshape(T, CD),
        num_mask.reshape(T, CD),
        W[NCAT * ED:],
        b.reshape(1, DM),
    )
    return out.reshape(B, S, DM)


# speedup vs baseline: 1.4222x; 1.4222x over previous
import functools

import jax
import jax.numpy as jnp
from jax import lax
from jax.experimental import pallas as pl
from jax.experimental.pallas import tpu as pltpu
from jax.experimental.pallas import tpu_sc as plsc

B, S, NCAT, V, ED, CD, DM = 4096, 50, 26, 100000, 32, 16, 128
T = B * S
NC4 = 28
RC = NC4 // 4
N2 = T * NC4
RPT = 128
NW = 32
BLOCKS = N2 // RPT
BPW = BLOCKS // NW
K = 14
OUTER = BPW // K


TT = 1600
TSTEPS = T // TT // NW


def _sc_idx_transpose(idx3):
    mesh = plsc.VectorSubcoreMesh(core_axis_name="c", subcore_axis_name="s")

    @functools.partial(
        pl.kernel,
        out_type=jax.ShapeDtypeStruct((RC, T, 4), jnp.int32),
        mesh=mesh,
        scratch_types=[
            pltpu.VMEM((TT, RC, 4), jnp.int32),
            pltpu.SemaphoreType.DMA,
        ],
        compiler_params=pltpu.CompilerParams(use_tc_tiling_on_sc=False),
    )
    def tr_kernel(idx_hbm, out_hbm, idx_v, sem):
        wid = lax.axis_index("s") * 2 + lax.axis_index("c")
        tbase = wid * (T // NW)

        def body(g, carry):
            t0 = tbase + g * TT
            pltpu.sync_copy(idx_hbm.at[pl.ds(t0, TT)], idx_v)
            copies = []
            for r in range(RC):
                copies.append(
                    pltpu.async_copy(
                        idx_v.at[:, r],
                        out_hbm.at[r].at[pl.ds(t0, TT)],
                        sem,
                    )
                )
            for c in copies:
                c.wait()
            return carry

        lax.fori_loop(0, TSTEPS, body, 0)

    return tr_kernel(idx3)


def _sc_gather(tables_flat, idx_flat):
    mesh = plsc.VectorSubcoreMesh(core_axis_name="c", subcore_axis_name="s")

    @functools.partial(
        pl.kernel,
        out_type=jax.ShapeDtypeStruct((N2, ED), jnp.float32),
        mesh=mesh,
        scratch_types=[
            pltpu.VMEM((K * RPT,), jnp.int32),
            pltpu.VMEM((K * RPT, ED), jnp.float32),
            pltpu.SemaphoreType.DMA,
        ],
        compiler_params=pltpu.CompilerParams(use_tc_tiling_on_sc=False),
    )
    def gather_kernel(tbl_hbm, idx_hbm, out_hbm, idx_v, rows_v, sem):
        wid = lax.axis_index("s") * 2 + lax.axis_index("c")
        wbase = wid * BPW

        def body(g, carry):
            row0 = (wbase + g * K) * RPT
            pltpu.sync_copy(idx_hbm.at[pl.ds(row0, K * RPT)], idx_v)
            copies = []
            for j in range(K):
                copies.append(
                    pltpu.async_copy(
                        tbl_hbm.at[idx_v.at[pl.ds(j * RPT, RPT)]],
                        rows_v.at[pl.ds(j * RPT, RPT)],
                        sem,
                    )
                )
            for c in copies:
                c.wait()
            pltpu.sync_copy(rows_v, out_hbm.at[pl.ds(row0, K * RPT)])
            return carry

        lax.fori_loop(0, OUTER, body, 0)

    return gather_kernel(tables_flat, idx_flat)


def _proj_body(x_ref, w3_ref, n_ref, m_ref, wn_ref, b_ref, o_ref):
    r = pl.program_id(1)

    @pl.when(r == 0)
    def _init():
        o_ref[...] = (
            jnp.dot(n_ref[...] * m_ref[...], wn_ref[...],
                    preferred_element_type=jnp.float32)
            + b_ref[...]
        )

    o_ref[...] += jnp.dot(x_ref[0], w3_ref[0],
                          preferred_element_type=jnp.float32)


def _tc_project(x3, w3, num_inputs, num_mask, Wn, b2):
    BT = 2048
    return pl.pallas_call(
        _proj_body,
        grid=(T // BT, RC),
        in_specs=[
            pl.BlockSpec((1, BT, 128), lambda i, r: (r, i, 0)),
            pl.BlockSpec((1, 128, DM), lambda i, r: (r, 0, 0)),
            pl.BlockSpec((BT, CD), lambda i, r: (i, 0)),
            pl.BlockSpec((BT, CD), lambda i, r: (i, 0)),
            pl.BlockSpec((CD, DM), lambda i, r: (0, 0)),
            pl.BlockSpec((1, DM), lambda i, r: (0, 0)),
        ],
        out_specs=pl.BlockSpec((BT, DM), lambda i, r: (i, 0)),
        out_shape=jax.ShapeDtypeStruct((T, DM), jnp.float32),
    )(x3, w3, num_inputs, num_mask, Wn, b2)


def kernel(cat_inputs, cat_mask, num_inputs, num_mask, tables, W, b):
    tables_flat = tables.reshape(NCAT * V, ED)
    offs = (jnp.arange(NCAT, dtype=jnp.int32) * V)[None, None, :]
    idx26 = (cat_inputs + offs).reshape(T, NCAT)
    idx28 = jnp.concatenate([idx26, idx26[:, :2]], axis=-1)

    idx_cm = _sc_idx_transpose(idx28.reshape(T, RC, 4))
    rows = _sc_gather(tables_flat, idx_cm.reshape(N2))
    x3 = rows.reshape(RC, T, 128)

    w3 = jnp.pad(W[: NCAT * ED], ((0, RC * 128 - NCAT * ED), (0, 0)))
    w3 = w3.reshape(RC, 128, DM)

    out = _tc_project(
        x3,
        w3,
        num_inputs.reshape(T, CD),
        num_mask.reshape(T, CD),
        W[NCAT * ED:],
        b.reshape(1, DM),
    )
    return out.reshape(B, S, DM)


# speedup vs baseline: 2.2991x; 1.6167x over previous
import functools

import jax
import jax.numpy as jnp
from jax import lax
from jax.experimental import pallas as pl
from jax.experimental.pallas import tpu as pltpu
from jax.experimental.pallas import tpu_sc as plsc

B, S, NCAT, V, ED, CD, DM = 4096, 50, 26, 100000, 32, 16, 128
T = B * S
N = T * NCAT
T4 = T // 4
RPT = 128
NW = 32
BLOCKS = N // RPT
BPC = T // RPT
BPW = BLOCKS // NW
K = 13
OUTER = BPW // K


def _sc_gather(tables, idx_cm):
    mesh = plsc.VectorSubcoreMesh(core_axis_name="c", subcore_axis_name="s")

    @functools.partial(
        pl.kernel,
        out_type=jax.ShapeDtypeStruct((N, ED), jnp.float32),
        mesh=mesh,
        scratch_types=[
            pltpu.VMEM((K * RPT,), jnp.int32),
            pltpu.VMEM((K * RPT, ED), jnp.float32),
            pltpu.SemaphoreType.DMA,
        ],
        compiler_params=pltpu.CompilerParams(use_tc_tiling_on_sc=False),
    )
    def gather_kernel(tbl_hbm, idx_hbm, out_hbm, idx_v, rows_v, sem):
        wid = lax.axis_index("s") * 2 + lax.axis_index("c")
        wbase = wid * BPW

        def body(g, carry):
            blk0 = wbase + g * K
            pltpu.sync_copy(
                idx_hbm.at[pl.ds(blk0 * RPT, K * RPT)], idx_v)
            copies = []
            for j in range(K):
                cat = (blk0 + j) // BPC
                copies.append(
                    pltpu.async_copy(
                        tbl_hbm.at[cat].at[idx_v.at[pl.ds(j * RPT, RPT)]],
                        rows_v.at[pl.ds(j * RPT, RPT)],
                        sem,
                    )
                )
            for c in copies:
                c.wait()
            pltpu.sync_copy(rows_v, out_hbm.at[pl.ds(blk0 * RPT, K * RPT)])
            return carry

        lax.fori_loop(0, OUTER, body, 0)

    return gather_kernel(tables, idx_cm.reshape(N))


def _proj_body(x_ref, wx_ref, n_ref, m_ref, wn_ref, b_ref, o_ref):
    c = pl.program_id(1)

    @pl.when(c == 0)
    def _init():
        nm = (n_ref[...] * m_ref[...]).astype(jnp.bfloat16)
        o_ref[...] = (
            jnp.dot(nm, wn_ref[...], preferred_element_type=jnp.float32)
            + b_ref[...]
        )

    o_ref[...] += jnp.dot(x_ref[0].astype(jnp.bfloat16), wx_ref[0],
                          preferred_element_type=jnp.float32)


def _tc_project(x26, wx, num4, mask4, wn4, b4):
    BT4 = 512
    return pl.pallas_call(
        _proj_body,
        grid=(T4 // BT4, NCAT),
        in_specs=[
            pl.BlockSpec((1, BT4, 128), lambda i, c: (c, i, 0)),
            pl.BlockSpec((1, 128, 512), lambda i, c: (c, 0, 0)),
            pl.BlockSpec((BT4, 4 * CD), lambda i, c: (i, 0)),
            pl.BlockSpec((BT4, 4 * CD), lambda i, c: (i, 0)),
            pl.BlockSpec((4 * CD, 512), lambda i, c: (0, 0)),
            pl.BlockSpec((1, 512), lambda i, c: (0, 0)),
        ],
        out_specs=pl.BlockSpec((BT4, 512), lambda i, c: (i, 0)),
        out_shape=jax.ShapeDtypeStruct((T4, 512), jnp.float32),
    )(x26, wx, num4, mask4, wn4, b4)


def kernel(cat_inputs, cat_mask, num_inputs, num_mask, tables, W, b):
    idx_cm = cat_inputs.transpose(2, 0, 1).reshape(NCAT, T)

    rows = _sc_gather(tables, idx_cm)
    x26 = rows.reshape(NCAT, T4, 128)

    eye4 = jnp.eye(4, dtype=jnp.float32)
    Wc = W[: NCAT * ED].reshape(NCAT, ED, DM)
    wx = (eye4[None, :, None, :, None] * Wc[:, None, :, None, :])
    wx = wx.reshape(NCAT, 4 * ED, 4 * DM).astype(jnp.bfloat16)
    Wn = W[NCAT * ED:]
    wn4 = (eye4[:, None, :, None] * Wn[None, :, None, :])
    wn4 = wn4.reshape(4 * CD, 4 * DM).astype(jnp.bfloat16)
    b4 = jnp.tile(b.reshape(1, DM), (1, 4))

    num4 = num_inputs.reshape(T4, 4 * CD)
    mask4 = num_mask.reshape(T4, 4 * CD)

    out4 = _tc_project(x26, wx, num4, mask4, wn4, b4)
    return out4.reshape(B, S, DM)


# speedup vs baseline: 3.6088x; 1.5696x over previous
import functools

import jax
import jax.numpy as jnp
from jax import lax
from jax.experimental import pallas as pl
from jax.experimental.pallas import tpu as pltpu
from jax.experimental.pallas import tpu_sc as plsc

B, S, NCAT, V, ED, CD, DM = 4096, 50, 26, 100000, 32, 16, 128
T = B * S
N = T * NCAT
T4 = T // 4
RPT = 128
NW = 32
BLOCKS = N // RPT
BPC = T // RPT
BPW = BLOCKS // NW
K = 13
OUTER = BPW // K


def _sc_gather(tables, idx_cm):
    mesh = plsc.VectorSubcoreMesh(core_axis_name="c", subcore_axis_name="s")

    @functools.partial(
        pl.kernel,
        out_type=jax.ShapeDtypeStruct((N, ED), jnp.float32),
        mesh=mesh,
        scratch_types=[
            pltpu.VMEM((K * RPT,), jnp.int32),
            pltpu.VMEM((K * RPT, ED), jnp.float32),
            pltpu.SemaphoreType.DMA,
        ],
        compiler_params=pltpu.CompilerParams(use_tc_tiling_on_sc=False),
    )
    def gather_kernel(tbl_hbm, idx_hbm, out_hbm, idx_v, rows_v, sem):
        wid = lax.axis_index("s") * 2 + lax.axis_index("c")
        wbase = wid * BPW

        def body(g, carry):
            blk0 = wbase + g * K
            pltpu.sync_copy(
                idx_hbm.at[pl.ds(blk0 * RPT, K * RPT)], idx_v)
            copies = []
            for j in range(K):
                cat = (blk0 + j) // BPC
                copies.append(
                    pltpu.async_copy(
                        tbl_hbm.at[cat].at[idx_v.at[pl.ds(j * RPT, RPT)]],
                        rows_v.at[pl.ds(j * RPT, RPT)],
                        sem,
                    )
                )
            for c in copies:
                c.wait()
            pltpu.sync_copy(rows_v, out_hbm.at[pl.ds(blk0 * RPT, K * RPT)])
            return carry

        lax.fori_loop(0, OUTER, body, 0)

    return gather_kernel(tables, idx_cm.reshape(N))


def _proj_body(x_ref, wx_ref, n_ref, m_ref, wn_ref, b_ref, o_ref):
    nm = (n_ref[...] * m_ref[...]).astype(jnp.bfloat16)
    acc = jnp.dot(nm, wn_ref[...], preferred_element_type=jnp.float32)
    for c in range(NCAT):
        acc += jnp.dot(x_ref[c].astype(jnp.bfloat16), wx_ref[c],
                       preferred_element_type=jnp.float32)
    o_ref[...] = acc + b_ref[...]


def _tc_project(x26, wx, num4, mask4, wn4, b4):
    BT4 = 512
    return pl.pallas_call(
        _proj_body,
        grid=(T4 // BT4,),
        in_specs=[
            pl.BlockSpec((NCAT, BT4, 128), lambda i: (0, i, 0)),
            pl.BlockSpec((NCAT, 128, 512), lambda i: (0, 0, 0)),
            pl.BlockSpec((BT4, 4 * CD), lambda i: (i, 0)),
            pl.BlockSpec((BT4, 4 * CD), lambda i: (i, 0)),
            pl.BlockSpec((4 * CD, 512), lambda i: (0, 0)),
            pl.BlockSpec((1, 512), lambda i: (0, 0)),
        ],
        out_specs=pl.BlockSpec((BT4, 512), lambda i: (i, 0)),
        out_shape=jax.ShapeDtypeStruct((T4, 512), jnp.float32),
    )(x26, wx, num4, mask4, wn4, b4)


def kernel(cat_inputs, cat_mask, num_inputs, num_mask, tables, W, b):
    idx_cm = cat_inputs.transpose(2, 0, 1).reshape(NCAT, T)

    rows = _sc_gather(tables, idx_cm)
    x26 = rows.reshape(NCAT, T4, 128)

    eye4 = jnp.eye(4, dtype=jnp.float32)
    Wc = W[: NCAT * ED].reshape(NCAT, ED, DM)
    wx = (eye4[None, :, None, :, None] * Wc[:, None, :, None, :])
    wx = wx.reshape(NCAT, 4 * ED, 4 * DM).astype(jnp.bfloat16)
    Wn = W[NCAT * ED:]
    wn4 = (eye4[:, None, :, None] * Wn[None, :, None, :])
    wn4 = wn4.reshape(4 * CD, 4 * DM).astype(jnp.bfloat16)
    b4 = jnp.tile(b.reshape(1, DM), (1, 4))

    num4 = num_inputs.reshape(T4, 4 * CD)
    mask4 = num_mask.reshape(T4, 4 * CD)

    out4 = _tc_project(x26, wx, num4, mask4, wn4, b4)
    return out4.reshape(B, S, DM)


# speedup vs baseline: 3.6094x; 1.0002x over previous
import functools

import jax
import jax.numpy as jnp
from jax import lax
from jax.experimental import pallas as pl
from jax.experimental.pallas import tpu as pltpu
from jax.experimental.pallas import tpu_sc as plsc

B, S, NCAT, V, ED, CD, DM = 4096, 50, 26, 100000, 32, 16, 128
T = B * S
N = T * NCAT
T4 = T // 4
RPT = 128
NW = 32
BLOCKS = N // RPT
BPC = T // RPT
BPW = BLOCKS // NW
K = 13
OUTER = BPW // K


def _sc_gather(tables_sub, idx_flat):
    mesh = plsc.VectorSubcoreMesh(core_axis_name="c", subcore_axis_name="s")

    @functools.partial(
        pl.kernel,
        out_type=jax.ShapeDtypeStruct((N, ED), jnp.float32),
        mesh=mesh,
        scratch_types=[
            pltpu.VMEM((K * RPT,), jnp.int32),
            pltpu.VMEM((K * RPT, ED), jnp.float32),
            pltpu.SemaphoreType.DMA,
        ],
        compiler_params=pltpu.CompilerParams(use_tc_tiling_on_sc=False),
    )
    def gather_kernel(tbl_hbm, idx_hbm, out_hbm, idx_v, rows_v, sem):
        wid = lax.axis_index("s") * 2 + lax.axis_index("c")
        wbase = wid * BPW

        def body(g, carry):
            blk0 = wbase + g * K
            pltpu.sync_copy(
                idx_hbm.at[pl.ds(blk0 * RPT, K * RPT)], idx_v)
            copies = []
            for j in range(K):
                copies.append(
                    pltpu.async_copy(
                        tbl_hbm.at[idx_v.at[pl.ds(j * RPT, RPT)]],
                        rows_v.at[pl.ds(j * RPT, RPT)],
                        sem,
                    )
                )
            for c in copies:
                c.wait()
            pltpu.sync_copy(rows_v, out_hbm.at[pl.ds(blk0 * RPT, K * RPT)])
            return carry

        lax.fori_loop(0, OUTER, body, 0)

    return gather_kernel(tables_sub, idx_flat)


def _proj_body(x_ref, wx_ref, n_ref, m_ref, wn_ref, b_ref, o_ref):
    nm = (n_ref[...] * m_ref[...]).astype(jnp.bfloat16)
    acc = jnp.dot(nm, wn_ref[...], preferred_element_type=jnp.float32)
    for c in range(NCAT):
        acc += jnp.dot(x_ref[c].astype(jnp.bfloat16), wx_ref[c],
                       preferred_element_type=jnp.float32)
    o_ref[...] = acc + b_ref[...]


def _tc_project(x26, wx, num4, mask4, wn4, b4):
    BT4 = 512
    return pl.pallas_call(
        _proj_body,
        grid=(T4 // BT4,),
        in_specs=[
            pl.BlockSpec((NCAT, BT4, 128), lambda i: (0, i, 0)),
            pl.BlockSpec((NCAT, 128, 512), lambda i: (0, 0, 0)),
            pl.BlockSpec((BT4, 4 * CD), lambda i: (i, 0)),
            pl.BlockSpec((BT4, 4 * CD), lambda i: (i, 0)),
            pl.BlockSpec((4 * CD, 512), lambda i: (0, 0)),
            pl.BlockSpec((1, 512), lambda i: (0, 0)),
        ],
        out_specs=pl.BlockSpec((BT4, 512), lambda i: (i, 0)),
        out_shape=jax.ShapeDtypeStruct((T4, 512), jnp.float32),
    )(x26, wx, num4, mask4, wn4, b4)


def kernel(cat_inputs, cat_mask, num_inputs, num_mask, tables, W, b):
    offs = (jnp.arange(NCAT, dtype=jnp.int32) * (4 * V))[:, None]
    idx_cm = cat_inputs.transpose(2, 0, 1).reshape(NCAT, T) * 4 + offs

    tables_pad = jnp.pad(tables, ((0, 0), (0, 0), (0, 3 * ED)))
    tables_sub = tables_pad.reshape(NCAT * V * 4, ED)

    rows = _sc_gather(tables_sub, idx_cm.reshape(N))
    x26 = rows.reshape(NCAT, T4, 128)

    eye4 = jnp.eye(4, dtype=jnp.float32)
    Wc = W[: NCAT * ED].reshape(NCAT, ED, DM)
    wx = (eye4[None, :, None, :, None] * Wc[:, None, :, None, :])
    wx = wx.reshape(NCAT, 4 * ED, 4 * DM).astype(jnp.bfloat16)
    Wn = W[NCAT * ED:]
    wn4 = (eye4[:, None, :, None] * Wn[None, :, None, :])
    wn4 = wn4.reshape(4 * CD, 4 * DM).astype(jnp.bfloat16)
    b4 = jnp.tile(b.reshape(1, DM), (1, 4))

    num4 = num_inputs.reshape(T4, 4 * CD)
    mask4 = num_mask.reshape(T4, 4 * CD)

    out4 = _tc_project(x26, wx, num4, mask4, wn4, b4)
    return out4.reshape(B, S, DM)
